# baseline (speedup 1.0000x reference)
import functools

import jax
import jax.numpy as jnp
from jax import lax
from jax.experimental import pallas as pl
from jax.experimental.pallas import tpu as pltpu
from jax.experimental.pallas import tpu_sc as plsc

N = 10000
D = 256
H = 128
NC = 2
NS = 16
K = 64
EPT = 160
E_PAD = NS * EPT * K
EPT_D = 80
ACC_ROWS = 10240
RB = 1000

_sc_mesh = plsc.VectorSubcoreMesh(
    core_axis_name="c", subcore_axis_name="s", num_cores=NC, num_subcores=NS)


_deg_kernel_args = dict(
    out_type=jax.ShapeDtypeStruct((NC * N, H), jnp.float32),
    mesh=_sc_mesh,
    scratch_types=[
        pltpu.VMEM((EPT_D, K), jnp.int32),
        pltpu.VMEM((K, H), jnp.float32),
        pltpu.VMEM_SHARED((ACC_ROWS, H), jnp.float32),
        pltpu.SemaphoreType.DMA,
    ],
)


def _sc_deg_body(dst_hbm, out_hbm, dstv, ones, acc, sem):
    c = lax.axis_index("c")
    s = lax.axis_index("s")
    pltpu.sync_copy(dst_hbm.at[c, s], dstv)

    def fill(val, i, carry):
        ones[i // 8, pl.ds((i % 8) * 16, 16)] = jnp.full((16,), val, jnp.float32)
        return carry
    lax.fori_loop(0, K * H // 16, functools.partial(fill, 0.0), None)
    for k in range(640 // K):
        pltpu.sync_copy(ones, acc.at[pl.ds(s * 640 + k * K, K)])
    lax.fori_loop(0, K * H // 16, functools.partial(fill, 1.0), None)
    plsc.subcore_barrier()

    def issue(j, carry):
        pltpu.async_copy(ones, acc.at[dstv.at[j]], sem, add=True)
        return carry
    lax.fori_loop(0, EPT_D, issue, None)

    def drain(j, carry):
        pltpu.make_async_copy(ones, acc.at[dstv.at[j]], sem).wait()
        return carry
    lax.fori_loop(0, EPT_D, drain, None)
    plsc.subcore_barrier()

    @pl.when(s < N // RB)
    def _():
        pltpu.sync_copy(acc.at[pl.ds(s * RB, RB)],
                        out_hbm.at[pl.ds(c * N + s * RB, RB)])


BLK = 64
_BLOCKS = ((0, BLK), (BLK, BLK), (2 * BLK, EPT - 2 * BLK))

_agg_kernel_args = dict(
    out_type=jax.ShapeDtypeStruct((NC * N, H), jnp.float32),
    mesh=_sc_mesh,
    scratch_types=[
        pltpu.VMEM((BLK, K), jnp.int32),
        pltpu.VMEM((BLK, K), jnp.int32),
        pltpu.VMEM((K, H), jnp.float32),
        pltpu.VMEM((K, H), jnp.float32),
        pltpu.VMEM((K, H), jnp.float32),
        pltpu.VMEM((K, H), jnp.float32),
        pltpu.VMEM_SHARED((ACC_ROWS, H), jnp.float32),
        pltpu.SemaphoreType.DMA,
        pltpu.SemaphoreType.DMA,
        pltpu.SemaphoreType.DMA,
        pltpu.SemaphoreType.DMA,
        pltpu.SemaphoreType.DMA,
        pltpu.SemaphoreType.DMA,
        pltpu.SemaphoreType.DMA,
        pltpu.SemaphoreType.DMA,
    ],
)

_NBUF = 4


def _sc_agg_body(hp_hbm, src_hbm, dst_hbm, out_hbm,
                 srcv, dstv, buf0, buf1, buf2, buf3, acc,
                 g0, g1, g2, g3, s0, s1, s2, s3):
    c = lax.axis_index("c")
    s = lax.axis_index("s")
    bufs = (buf0, buf1, buf2, buf3)
    gsem = (g0, g1, g2, g3)
    ssem = (s0, s1, s2, s3)

    def _gather(j, b):
        pltpu.async_copy(hp_hbm.at[srcv.at[j]], bufs[b], gsem[b])

    def _wait_gather(j, b):
        pltpu.make_async_copy(hp_hbm.at[srcv.at[j]], bufs[b], gsem[b]).wait()

    def _scatter(j, b):
        pltpu.async_copy(bufs[b], acc.at[dstv.at[j]], ssem[b], add=True)

    def _wait_scatter(j, b):
        pltpu.make_async_copy(bufs[b], acc.at[dstv.at[j]], ssem[b]).wait()

    def zb(i, carry):
        buf0[i // 8, pl.ds((i % 8) * 16, 16)] = jnp.zeros((16,), jnp.float32)
        return carry
    lax.fori_loop(0, K * H // 16, zb, None)
    for k in range(640 // K):
        pltpu.sync_copy(buf0, acc.at[pl.ds(s * 640 + k * K, K)])
    plsc.subcore_barrier()

    for base, nb in _BLOCKS:
        m = nb // _NBUF
        pltpu.sync_copy(src_hbm.at[c, s, pl.ds(base, nb)],
                        srcv.at[pl.ds(0, nb)])
        pltpu.sync_copy(dst_hbm.at[s, pl.ds(base, nb)],
                        dstv.at[pl.ds(0, nb)])
        for b in range(_NBUF):
            _gather(b, b)
        for b in range(_NBUF):
            _wait_gather(b, b)
            _scatter(b, b)
            if b >= 2:
                _wait_scatter(b - 2, b - 2)
                _gather(b + 2, b - 2)

        def body(i, carry):
            for b in range(_NBUF):
                j = _NBUF * i + b
                _wait_gather(j, b)
                _scatter(j, b)
                b2 = (b + 2) % _NBUF
                _wait_scatter(j - 2, b2)
                _gather(j + 2, b2)
            return carry
        lax.fori_loop(1, m - 1, body, None)
        for b in range(_NBUF):
            j = nb - _NBUF + b
            _wait_gather(j, b)
            _scatter(j, b)
            if b < 2:
                _wait_scatter(j - 2, b + 2)
                _gather(j + 2, b + 2)
        for b in range(_NBUF):
            _wait_scatter(nb - _NBUF + b, b)
    plsc.subcore_barrier()

    @pl.when(s < N // RB)
    def _():
        pltpu.sync_copy(acc.at[pl.ds(s * RB, RB)],
                        out_hbm.at[pl.ds(c * N + s * RB, RB)])


_sc_deg = pl.kernel(_sc_deg_body, **_deg_kernel_args)
_sc_agg = pl.kernel(_sc_agg_body, **_agg_kernel_args)


def _dinv_of(deg_ref):
    d = deg_ref[:, 0:1]
    return jnp.where(d > 0.0, lax.rsqrt(d), 0.0)


def _tc_first_body(x_ref, w_ref, deg_ref, hp_ref):
    dinv = _dinv_of(deg_ref)
    h = jnp.dot(x_ref[...], w_ref[...],
                preferred_element_type=jnp.float32) * dinv
    hp_ref[0] = h[:, :H]
    hp_ref[1] = h[:, H:]


def _tc_mid_body(a_ref, deg_ref, b_ref, w_ref, h_ref, hp_ref):
    dinv = _dinv_of(deg_ref)
    agg = jnp.concatenate([a_ref[0], a_ref[1]], axis=1) * dinv
    hl = jnp.maximum(agg + b_ref[...], 0.0)
    h_ref[...] = hl
    hp = jnp.dot(hl, w_ref[...], preferred_element_type=jnp.float32) * dinv
    hp_ref[0] = hp[:, :H]
    hp_ref[1] = hp[:, H:]


def _tc_last_body(a_ref, deg_ref, b_ref, h_ref):
    dinv = _dinv_of(deg_ref)
    agg = jnp.concatenate([a_ref[0], a_ref[1]], axis=1) * dinv
    h_ref[...] = jnp.maximum(agg + b_ref[...], 0.0)


_spec_rows = pl.BlockSpec((RB, D), lambda i: (i, 0))
_spec_w = pl.BlockSpec((D, D), lambda i: (0, 0))
_spec_deg = pl.BlockSpec((RB, 16), lambda i: (i, 0))
_spec_b = pl.BlockSpec((1, D), lambda i: (0, 0))
_spec_hp = pl.BlockSpec((2, RB, H), lambda i: (0, i, 0))

_tc_first = pl.pallas_call(
    _tc_first_body,
    grid=(N // RB,),
    in_specs=[_spec_rows, _spec_w, _spec_deg],
    out_specs=_spec_hp,
    out_shape=jax.ShapeDtypeStruct((2, N, H), jnp.float32),
)

_tc_mid = pl.pallas_call(
    _tc_mid_body,
    grid=(N // RB,),
    in_specs=[_spec_hp, _spec_deg, _spec_b, _spec_w],
    out_specs=(_spec_rows, _spec_hp),
    out_shape=(jax.ShapeDtypeStruct((N, D), jnp.float32),
               jax.ShapeDtypeStruct((2, N, H), jnp.float32)),
)

_tc_last = pl.pallas_call(
    _tc_last_body,
    grid=(N // RB,),
    in_specs=[_spec_hp, _spec_deg, _spec_b],
    out_specs=_spec_rows,
    out_shape=jax.ShapeDtypeStruct((N, D), jnp.float32),
)


def kernel(x, adj_t, W1, b1, W2, b2, W3, b3):
    src = adj_t[0]
    dst = adj_t[1]
    e = src.shape[0]
    pad = E_PAD - e
    src_p = jnp.concatenate(
        [src, jnp.arange(pad, dtype=jnp.int32) % jnp.int32(N)])
    dst_p = jnp.concatenate([dst, jnp.full((pad,), N, jnp.int32)])
    src2 = jnp.stack([src_p, src_p + N]).reshape(NC, NS, EPT, K)
    dst3 = dst_p.reshape(NS, EPT, K)
    dst_d = dst_p.reshape(NC, NS, EPT_D, K)
    dpart = _sc_deg(dst_d)
    deg16 = dpart[:N, :16] + dpart[N:, :16]
    b1r, b2r, b3r = (b.reshape(1, D) for b in (b1, b2, b3))

    def agg(hp):
        return _sc_agg(hp.reshape(NC * N, H), src2, dst3).reshape(NC, N, H)

    hp1 = _tc_first(x, W1, deg16)
    h1, hp2 = _tc_mid(agg(hp1), deg16, b1r, W2)
    h2, hp3 = _tc_mid(agg(hp2), deg16, b2r, W3)
    h3 = _tc_last(agg(hp3), deg16, b3r)
    return (h1, h2, h3)


# speedup vs baseline: 1.1431x; 1.1431x over previous
import functools

import jax
import jax.numpy as jnp
from jax import lax
from jax.experimental import pallas as pl
from jax.experimental.pallas import tpu as pltpu
from jax.experimental.pallas import tpu_sc as plsc

N = 10000
D = 256
H = 128
NC = 2
NS = 16
K = 64
EPT = 160
E_PAD = NS * EPT * K
EPT_D = 80
ACC_ROWS = 10240
RB = 1000

_sc_mesh = plsc.VectorSubcoreMesh(
    core_axis_name="c", subcore_axis_name="s", num_cores=NC, num_subcores=NS)


_deg_kernel_args = dict(
    out_type=jax.ShapeDtypeStruct((NC * N, H), jnp.float32),
    mesh=_sc_mesh,
    scratch_types=[
        pltpu.VMEM((EPT_D, K), jnp.int32),
        pltpu.VMEM((K, H), jnp.float32),
        pltpu.VMEM_SHARED((ACC_ROWS, H), jnp.float32),
        pltpu.SemaphoreType.DMA,
    ],
)


def _sc_deg_body(dst_hbm, out_hbm, dstv, ones, acc, sem):
    c = lax.axis_index("c")
    s = lax.axis_index("s")
    pltpu.sync_copy(dst_hbm.at[c, s], dstv)

    def fill(val, i, carry):
        ones[i // 8, pl.ds((i % 8) * 16, 16)] = jnp.full((16,), val, jnp.float32)
        return carry
    lax.fori_loop(0, K * H // 16, functools.partial(fill, 0.0), None)
    for k in range(640 // K):
        pltpu.sync_copy(ones, acc.at[pl.ds(s * 640 + k * K, K)])
    lax.fori_loop(0, K * H // 16, functools.partial(fill, 1.0), None)
    plsc.subcore_barrier()

    def issue(j, carry):
        pltpu.async_copy(ones, acc.at[dstv.at[j]], sem, add=True)
        return carry
    lax.fori_loop(0, EPT_D, issue, None)

    def drain(j, carry):
        pltpu.make_async_copy(ones, acc.at[dstv.at[j]], sem).wait()
        return carry
    lax.fori_loop(0, EPT_D, drain, None)
    plsc.subcore_barrier()

    @pl.when(s < N // RB)
    def _():
        pltpu.sync_copy(acc.at[pl.ds(s * RB, RB)],
                        out_hbm.at[pl.ds(c * N + s * RB, RB)])


BLK = 64
_BLOCKS = ((0, BLK), (BLK, BLK), (2 * BLK, EPT - 2 * BLK))

_agg_kernel_args = dict(
    out_type=jax.ShapeDtypeStruct((NC * N, H), jnp.float32),
    mesh=_sc_mesh,
    scratch_types=[
        pltpu.VMEM((BLK, K), jnp.int32),
        pltpu.VMEM((BLK, K), jnp.int32),
        pltpu.VMEM((K, H), jnp.float32),
        pltpu.VMEM((K, H), jnp.float32),
        pltpu.VMEM((K, H), jnp.float32),
        pltpu.VMEM((K, H), jnp.float32),
        pltpu.VMEM_SHARED((ACC_ROWS, H), jnp.float32),
        pltpu.SemaphoreType.DMA,
        pltpu.SemaphoreType.DMA,
        pltpu.SemaphoreType.DMA,
        pltpu.SemaphoreType.DMA,
    ],
)

_NBUF = 4


def _sc_agg_body(hp_hbm, src_hbm, dst_hbm, out_hbm,
                 srcv, dstv, buf0, buf1, buf2, buf3, acc,
                 sem0, sem1, sem2, sem3):
    c = lax.axis_index("c")
    s = lax.axis_index("s")
    bufs = ((buf0, sem0), (buf1, sem1), (buf2, sem2), (buf3, sem3))

    def zb(i, carry):
        buf0[i // 8, pl.ds((i % 8) * 16, 16)] = jnp.zeros((16,), jnp.float32)
        return carry
    lax.fori_loop(0, K * H // 16, zb, None)
    for k in range(640 // K):
        pltpu.sync_copy(buf0, acc.at[pl.ds(s * 640 + k * K, K)])
    plsc.subcore_barrier()

    for base, nb in _BLOCKS:
        m = _NBUF * (nb // _NBUF)
        pltpu.sync_copy(src_hbm.at[c, s, pl.ds(base, nb)],
                        srcv.at[pl.ds(0, nb)])
        pltpu.sync_copy(dst_hbm.at[s, pl.ds(base, nb)],
                        dstv.at[pl.ds(0, nb)])
        for b, (buf, sem) in enumerate(bufs):
            pltpu.async_copy(hp_hbm.at[srcv.at[b]], buf, sem)

        def body(i, carry, nb=nb):
            for b, (buf, sem) in enumerate(bufs):
                j = _NBUF * i + b
                pltpu.make_async_copy(hp_hbm.at[srcv.at[j]], buf, sem).wait()
                pltpu.sync_copy(buf, acc.at[dstv.at[j]], add=True)
                jn = jnp.minimum(j + _NBUF, nb - 1)
                pltpu.async_copy(hp_hbm.at[srcv.at[jn]], buf, sem)
            return carry
        lax.fori_loop(0, nb // _NBUF, body, None)
        for j in range(m, nb):
            buf, sem = bufs[j - m]
            pltpu.make_async_copy(hp_hbm.at[srcv.at[j]], buf, sem).wait()
            pltpu.sync_copy(buf, acc.at[dstv.at[j]], add=True)
        for b in range(nb - m, _NBUF):
            buf, sem = bufs[b]
            pltpu.make_async_copy(hp_hbm.at[srcv.at[nb - 1]], buf, sem).wait()
    plsc.subcore_barrier()

    @pl.when(s < N // RB)
    def _():
        pltpu.sync_copy(acc.at[pl.ds(s * RB, RB)],
                        out_hbm.at[pl.ds(c * N + s * RB, RB)])


_sc_deg = pl.kernel(_sc_deg_body, **_deg_kernel_args)
_sc_agg = pl.kernel(_sc_agg_body, **_agg_kernel_args)


def _dinv_of(deg_ref):
    d = deg_ref[:, 0:1]
    return jnp.where(d > 0.0, lax.rsqrt(d), 0.0)


def _tc_first_body(x_ref, w_ref, deg_ref, hp_ref):
    dinv = _dinv_of(deg_ref)
    h = jnp.dot(x_ref[...], w_ref[...],
                preferred_element_type=jnp.float32) * dinv
    hp_ref[0] = h[:, :H]
    hp_ref[1] = h[:, H:]


def _tc_mid_body(a_ref, deg_ref, b_ref, w_ref, h_ref, hp_ref):
    dinv = _dinv_of(deg_ref)
    agg = jnp.concatenate([a_ref[0], a_ref[1]], axis=1) * dinv
    hl = jnp.maximum(agg + b_ref[...], 0.0)
    h_ref[...] = hl
    hp = jnp.dot(hl, w_ref[...], preferred_element_type=jnp.float32) * dinv
    hp_ref[0] = hp[:, :H]
    hp_ref[1] = hp[:, H:]


def _tc_last_body(a_ref, deg_ref, b_ref, h_ref):
    dinv = _dinv_of(deg_ref)
    agg = jnp.concatenate([a_ref[0], a_ref[1]], axis=1) * dinv
    h_ref[...] = jnp.maximum(agg + b_ref[...], 0.0)


_spec_rows = pl.BlockSpec((RB, D), lambda i: (i, 0))
_spec_w = pl.BlockSpec((D, D), lambda i: (0, 0))
_spec_deg = pl.BlockSpec((RB, 16), lambda i: (i, 0))
_spec_b = pl.BlockSpec((1, D), lambda i: (0, 0))
_spec_hp = pl.BlockSpec((2, RB, H), lambda i: (0, i, 0))

_tc_first = pl.pallas_call(
    _tc_first_body,
    grid=(N // RB,),
    in_specs=[_spec_rows, _spec_w, _spec_deg],
    out_specs=_spec_hp,
    out_shape=jax.ShapeDtypeStruct((2, N, H), jnp.float32),
)

_tc_mid = pl.pallas_call(
    _tc_mid_body,
    grid=(N // RB,),
    in_specs=[_spec_hp, _spec_deg, _spec_b, _spec_w],
    out_specs=(_spec_rows, _spec_hp),
    out_shape=(jax.ShapeDtypeStruct((N, D), jnp.float32),
               jax.ShapeDtypeStruct((2, N, H), jnp.float32)),
)

_tc_last = pl.pallas_call(
    _tc_last_body,
    grid=(N // RB,),
    in_specs=[_spec_hp, _spec_deg, _spec_b],
    out_specs=_spec_rows,
    out_shape=jax.ShapeDtypeStruct((N, D), jnp.float32),
)


def kernel(x, adj_t, W1, b1, W2, b2, W3, b3):
    src = adj_t[0]
    dst = adj_t[1]
    e = src.shape[0]
    pad = E_PAD - e
    src_p = jnp.concatenate(
        [src, jnp.arange(pad, dtype=jnp.int32) % jnp.int32(N)])
    dst_p = jnp.concatenate([dst, jnp.full((pad,), N, jnp.int32)])
    src2 = jnp.stack([src_p, src_p + N]).reshape(NC, NS, EPT, K)
    dst3 = dst_p.reshape(NS, EPT, K)
    dst_d = dst_p.reshape(NC, NS, EPT_D, K)
    dpart = _sc_deg(dst_d)
    deg16 = dpart[:N, :16] + dpart[N:, :16]
    b1r, b2r, b3r = (b.reshape(1, D) for b in (b1, b2, b3))

    def agg(hp):
        return _sc_agg(hp.reshape(NC * N, H), src2, dst3).reshape(NC, N, H)

    hp1 = _tc_first(x, W1, deg16)
    h1, hp2 = _tc_mid(agg(hp1), deg16, b1r, W2)
    h2, hp3 = _tc_mid(agg(hp2), deg16, b2r, W3)
    h3 = _tc_last(agg(hp3), deg16, b3r)
    return (h1, h2, h3)
